# baseline (speedup 1.0000x reference)
import functools

import jax
import jax.numpy as jnp
from jax import lax
from jax.experimental import pallas as pl
from jax.experimental.pallas import tpu as pltpu
from jax.experimental.pallas import tpu_sc as plsc

jax.config.update("jax_enable_x64", True)

_EMD_DIM = 32
_NUM_V = 1000000
_MAX_STEPS = 22
_L = 16
_WIN = 128
_N_WROWS = _NUM_V * _EMD_DIM // _WIN


def _log1p_poly(y):
    s = y / (y + 2.0)
    s2 = s * s
    p = jnp.float32(1.0 / 11.0)
    for c in (1.0 / 9.0, 1.0 / 7.0, 1.0 / 5.0, 1.0 / 3.0, 1.0):
        p = p * s2 + jnp.float32(c)
    return 2.0 * s * p


def _sc_body(uv_hbm, vv_hbm, emd_hbm, h_hbm, out_hbm,
             uv_v, vv_v, hidx_v, eidx_v, hrows_v, erows_v, prod_v,
             sem_h, sem_e):
    cid = lax.axis_index("c")
    sid = lax.axis_index("s")

    @pl.when(jnp.logical_and(cid == 0, sid == 0))
    def _():
        pltpu.sync_copy(uv_hbm, uv_v)
        pltpu.sync_copy(vv_hbm, vv_v)
        v = vv_v[...]
        u = uv_v[...]

        e_wrow = jnp.minimum(v >> 2, _N_WROWS - 1)
        o_e = (v << 5) - (e_wrow << 7)
        eidx_v[...] = e_wrow
        e_cp = pltpu.make_async_copy(emd_hbm.at[eidx_v], erows_v, sem_e)
        e_cp.start()

        lane = lax.iota(jnp.int32, _L)
        t = 2 * (_NUM_V - 1 + u)
        n = jnp.zeros((_L,), jnp.int32)
        idx_a = jnp.zeros((_L,), jnp.int32)
        idx_b = jnp.zeros((_L,), jnp.int32)
        off_a = jnp.zeros((_L,), jnp.int32)
        off_b = jnp.zeros((_L,), jnp.int32)
        for k in range(_MAX_STEPS):
            active = t != 0
            n = n + jnp.where(active, jnp.int32(1), jnp.int32(0))
            t_raw = jnp.where((t & 3) == 0, (t >> 1) - 1, t >> 1)
            t = jnp.where(active, t_raw, 0)
            i = t >> 1
            wrow = jnp.minimum(i >> 2, _N_WROWS - 1)
            off = (i << 5) - (wrow << 7)
            if k < _L:
                idx_a = jnp.where(lane == k, wrow, idx_a)
                off_a = jnp.where(lane == k, off, off_a)
            else:
                idx_b = jnp.where(lane == (k - _L), wrow, idx_b)
                off_b = jnp.where(lane == (k - _L), off, off_b)
        hidx_v[pl.ds(0, _L)] = idx_a
        hidx_v[pl.ds(_L, _L)] = idx_b

        h_cp = pltpu.make_async_copy(h_hbm.at[hidx_v], hrows_v, sem_h)
        h_cp.start()
        e_cp.wait()
        h_cp.wait()

        zero16 = jnp.zeros((_L,), jnp.int32)
        acc_a = jnp.zeros((_L,), jnp.float32)
        acc_b = jnp.zeros((_L,), jnp.float32)
        for j in range(_EMD_DIM):
            ej = plsc.load_gather(erows_v, [zero16, o_e + j])
            col_a = plsc.load_gather(hrows_v, [lane, off_a + j])
            col_b = plsc.load_gather(hrows_v, [lane + _L, off_b + j])
            acc_a = acc_a + ej * col_a
            acc_b = acc_b + ej * col_b

        def logsig(d):
            return jnp.minimum(d, 0.0) - _log1p_poly(jnp.exp(-jnp.abs(d)))

        f_a = jnp.where(lane < n, logsig(acc_a), jnp.float32(1.0))
        f_b = jnp.where(lane + _L < n, logsig(acc_b), jnp.float32(1.0))
        f = f_a * f_b
        for half in (8, 4, 2, 1):
            prod_v[...] = f
            g = plsc.load_gather(prod_v, [lane ^ half])
            f = f * g
        prod_v[...] = -f
        pltpu.sync_copy(prod_v, out_hbm)


def _sc_call(uv, vv, e2, h2):
    mesh = plsc.VectorSubcoreMesh(core_axis_name="c", subcore_axis_name="s")
    fn = functools.partial(
        pl.kernel,
        out_type=jax.ShapeDtypeStruct((_L,), jnp.float32),
        mesh=mesh,
        compiler_params=pltpu.CompilerParams(
            use_tc_tiling_on_sc=False, needs_layout_passes=False
        ),
        scratch_types=[
            pltpu.VMEM((_L,), jnp.int32),
            pltpu.VMEM((_L,), jnp.int32),
            pltpu.VMEM((2 * _L,), jnp.int32),
            pltpu.VMEM((_L,), jnp.int32),
            pltpu.VMEM((2 * _L, _WIN), jnp.float32),
            pltpu.VMEM((_L, _WIN), jnp.float32),
            pltpu.VMEM((_L,), jnp.float32),
            pltpu.SemaphoreType.DMA,
            pltpu.SemaphoreType.DMA,
        ],
    )(_sc_body)
    return fn(uv, vv, e2, h2)


def kernel(v_j, u_k, emd_table, h_table):
    e2 = emd_table.astype(jnp.float32).reshape(_N_WROWS, _WIN)
    h_flat = h_table.astype(jnp.float32).reshape(-1)
    h2 = jnp.concatenate([h_flat, jnp.zeros(_EMD_DIM, jnp.float32)]).reshape(
        _N_WROWS, _WIN
    )
    uv = jnp.full((_L,), u_k, jnp.int32)
    vv = jnp.full((_L,), v_j, jnp.int32)
    out = _sc_call(uv, vv, e2, h2)
    return out[0].astype(jnp.float64)
